# baseline (speedup 1.0000x reference)
import functools
import jax
import jax.numpy as jnp
from jax.experimental import pallas as pl
from jax.experimental.pallas import tpu as pltpu

DIM = 96
H = 224
W = 224
WS = 8
HEADS = 8
TOPK = 4
HIDDEN = 192
NH = H // WS
NWIN = NH * NH
WS2 = WS * WS
L = H * W
DH = DIM // HEADS
ROWS_PER_STEP = WS * W


def _ln_qkv_body(x_ref, g1_ref, b1_ref, wqkv_ref, bqkv_ref,
                 qsp_ref, kw_ref, vw_ref, qr_ref, kr_ref):
    x = x_ref[...].reshape(DIM, ROWS_PER_STEP).T
    mu = jnp.mean(x, axis=-1, keepdims=True)
    var = jnp.mean((x - mu) ** 2, axis=-1, keepdims=True)
    y = (x - mu) * jax.lax.rsqrt(var + 1e-5) * g1_ref[...] + b1_ref[...]
    qkv = jnp.dot(y, wqkv_ref[...], preferred_element_type=jnp.float32)
    qkv = qkv + bqkv_ref[...]
    q = qkv[:, :DIM]
    k = qkv[:, 128:128 + DIM]
    v = qkv[:, 256:256 + DIM]
    qsp_ref[...] = q.reshape(WS, W, DIM)
    k4 = k.reshape(WS, NH, WS, DIM)
    v4 = v.reshape(WS, NH, WS, DIM)
    kw_ref[...] = k4.transpose(1, 0, 2, 3).reshape(NH, WS2, DIM)
    vw_ref[...] = v4.transpose(1, 0, 2, 3).reshape(NH, WS2, DIM)
    qr_ref[0] = jnp.mean(q.reshape(WS, NH, WS, DIM), axis=(0, 2))
    kr_ref[0] = jnp.mean(k4, axis=(0, 2))


def _ln_qkv(x, g1, b1, wqkv_pad, bqkv_pad):
    return pl.pallas_call(
        _ln_qkv_body,
        grid=(NH,),
        in_specs=[
            pl.BlockSpec((1, DIM, WS, W), lambda i: (0, 0, i, 0)),
            pl.BlockSpec((DIM,), lambda i: (0,)),
            pl.BlockSpec((DIM,), lambda i: (0,)),
            pl.BlockSpec((DIM, 384), lambda i: (0, 0)),
            pl.BlockSpec((384,), lambda i: (0,)),
        ],
        out_specs=[
            pl.BlockSpec((WS, W, DIM), lambda i: (i, 0, 0)),
            pl.BlockSpec((NH, WS2, DIM), lambda i: (i, 0, 0)),
            pl.BlockSpec((NH, WS2, DIM), lambda i: (i, 0, 0)),
            pl.BlockSpec((1, NH, DIM), lambda i: (i, 0, 0)),
            pl.BlockSpec((1, NH, DIM), lambda i: (i, 0, 0)),
        ],
        out_shape=[
            jax.ShapeDtypeStruct((H, W, DIM), jnp.float32),
            jax.ShapeDtypeStruct((NWIN, WS2, DIM), jnp.float32),
            jax.ShapeDtypeStruct((NWIN, WS2, DIM), jnp.float32),
            jax.ShapeDtypeStruct((NH, NH, DIM), jnp.float32),
            jax.ShapeDtypeStruct((NH, NH, DIM), jnp.float32),
        ],
    )(x, g1, b1, wqkv_pad, bqkv_pad)


def _route_body(qr_ref, kr_ref, idx_ref):
    qr = qr_ref[...].reshape(NWIN, DIM)
    kr = kr_ref[...].reshape(NWIN, DIM)
    a = jnp.dot(qr, kr.T, preferred_element_type=jnp.float32)
    col = jax.lax.broadcasted_iota(jnp.int32, (NWIN, NWIN), 1)
    for t in range(TOPK):
        m = jnp.max(a, axis=-1, keepdims=True)
        amax = jnp.min(jnp.where(a >= m, col, jnp.int32(2 ** 30)),
                       axis=-1, keepdims=True)
        idx_ref[t, :] = amax[:, 0]
        a = jnp.where(col == amax, -jnp.inf, a)


def _route(qr, kr):
    return pl.pallas_call(
        _route_body,
        grid=(1,),
        in_specs=[
            pl.BlockSpec((NH, NH, DIM), lambda i: (0, 0, 0)),
            pl.BlockSpec((NH, NH, DIM), lambda i: (0, 0, 0)),
        ],
        out_specs=pl.BlockSpec((8, NWIN), lambda i: (0, 0)),
        out_shape=jax.ShapeDtypeStruct((8, NWIN), jnp.int32),
    )(qr, kr)


WIN_PER_STEP = 28
SCALE = DH ** -0.5


def _win_rows_to_spatial(t, nrows):
    return (t.reshape(NH, nrows, WS, DIM)
             .transpose(1, 0, 2, 3)
             .reshape(nrows, W, DIM))


UNROLL = 4


def _attn_lepe_body(idx_ref, qw_ref, kw_ref, vw_ref, x_ref,
                    lw_ref, wo_ref, bo_ref, x1_ref, obuf_ref):
    step = pl.program_id(0)
    r = jax.lax.broadcasted_iota(jnp.int32, (HEADS * WS2, DIM), 0)
    c = jax.lax.broadcasted_iota(jnp.int32, (HEADS * WS2, DIM), 1)
    head_mask = (r // WS2) == (c // DH)

    def win_block(it, carry):
        for u in range(UNROLL):
            wloc = it * UNROLL + u
            p = step * NH + wloc
            kparts = []
            vparts = []
            for t in range(TOPK):
                wi = idx_ref[t, p]
                kparts.append(kw_ref[wi])
                vparts.append(vw_ref[wi])
            kg = jnp.concatenate(kparts, axis=0)
            vg = jnp.concatenate(vparts, axis=0)
            q = qw_ref[:, pl.ds(wloc * WS, WS), :].reshape(WS2, DIM)
            q8 = jnp.concatenate([q] * HEADS, axis=0)
            qbd = jnp.where(head_mask, q8, 0.0) * SCALE
            s = jax.lax.dot_general(
                qbd, kg, (((1,), (1,)), ((), ())),
                preferred_element_type=jnp.float32)
            e = jnp.exp(s)
            rden = 1.0 / jnp.sum(e, axis=-1, keepdims=True)
            o_all = jax.lax.dot_general(
                e, vg, (((1,), (0,)), ((), ())),
                preferred_element_type=jnp.float32) * rden
            o = jnp.concatenate(
                [o_all[h * WS2:(h + 1) * WS2, h * DH:(h + 1) * DH]
                 for h in range(HEADS)], axis=1)
            obuf_ref[:, pl.ds(wloc * WS, WS), :] = o.reshape(WS, WS, DIM)
        return carry

    jax.lax.fori_loop(0, WIN_PER_STEP // UNROLL, win_block, 0)
    o_sp = obuf_ref[...]

    i = step
    vc = _win_rows_to_spatial(
        vw_ref[pl.ds(i * NH, NH)], WS)
    vp = _win_rows_to_spatial(
        vw_ref[pl.ds(jnp.maximum(i - 1, 0) * NH, NH)][:, WS2 - 2 * WS:, :], 2)
    vn = _win_rows_to_spatial(
        vw_ref[pl.ds(jnp.minimum(i + 1, NH - 1) * NH, NH)][:, :2 * WS, :], 2)
    top = jnp.where(i > 0, vp, 0.0)
    bot = jnp.where(i < NH - 1, vn, 0.0)
    vpad = jnp.concatenate([top, vc, bot], axis=0)
    zc = jnp.zeros((WS + 4, 2, DIM), jnp.float32)
    vpad = jnp.concatenate([zc, vpad, zc], axis=1)
    lw = lw_ref[...]
    lepe = jnp.zeros((WS, W, DIM), jnp.float32)
    for b in range(5):
        sb = vpad[:, b:b + W, :]
        for a in range(5):
            lepe = lepe + sb[a:a + WS] * lw[a * 5 + b]
    o = o_sp + lepe
    y = jnp.dot(o.reshape(ROWS_PER_STEP, DIM), wo_ref[...],
                preferred_element_type=jnp.float32) + bo_ref[...]
    xl = x_ref[...].reshape(DIM, ROWS_PER_STEP).T
    x1_ref[...] = xl + y


def _attn_lepe(idx8, q_sp, kw, vw, x, lepe_wt, wo, bo):
    return pl.pallas_call(
        _attn_lepe_body,
        grid_spec=pltpu.PrefetchScalarGridSpec(
            num_scalar_prefetch=1,
            grid=(NWIN // WIN_PER_STEP,),
            in_specs=[
                pl.BlockSpec((WS, W, DIM), lambda i, idx: (i, 0, 0)),
                pl.BlockSpec((NWIN, WS2, DIM), lambda i, idx: (0, 0, 0)),
                pl.BlockSpec((NWIN, WS2, DIM), lambda i, idx: (0, 0, 0)),
                pl.BlockSpec((1, DIM, WS, W), lambda i, idx: (0, 0, i, 0)),
                pl.BlockSpec((25, DIM), lambda i, idx: (0, 0)),
                pl.BlockSpec((DIM, DIM), lambda i, idx: (0, 0)),
                pl.BlockSpec((DIM,), lambda i, idx: (0,)),
            ],
            out_specs=pl.BlockSpec((ROWS_PER_STEP, DIM),
                                   lambda i, idx: (i, 0)),
            scratch_shapes=[pltpu.VMEM((WS, W, DIM), jnp.float32)],
        ),
        out_shape=jax.ShapeDtypeStruct((L, DIM), jnp.float32),
        compiler_params=pltpu.CompilerParams(
            dimension_semantics=("arbitrary",)),
    )(idx8, q_sp, kw, vw, x, lepe_wt, wo, bo)


def _mlp_body(xp_ref, xc_ref, xn_ref, g2_ref, b2_ref,
              wint_ref, wdw_ref, woutt_ref, y_ref):
    i = pl.program_id(0)
    xc = xc_ref[...]
    xh = jnp.concatenate(
        [xp_ref[WS - 1:WS], xc, xn_ref[0:1]], axis=0)
    mu = jnp.mean(xh, axis=-1, keepdims=True)
    var = jnp.mean((xh - mu) ** 2, axis=-1, keepdims=True)
    y2 = (xh - mu) * jax.lax.rsqrt(var + 1e-5) * g2_ref[...] + b2_ref[...]
    t = jnp.dot(y2.reshape((WS + 2) * W, DIM), wint_ref[...],
                preferred_element_type=jnp.float32)
    t = t.reshape(WS + 2, W, HIDDEN)
    row = jax.lax.broadcasted_iota(jnp.int32, (WS + 2, 1, 1), 0)
    valid = jnp.logical_and(
        jnp.logical_or(i > 0, row > 0),
        jnp.logical_or(i < NH - 1, row < WS + 1))
    t = jnp.where(valid, t, 0.0)
    zc = jnp.zeros((WS + 2, 1, HIDDEN), jnp.float32)
    t = jnp.concatenate([zc, t, zc], axis=1)
    wdw = wdw_ref[...]
    acc = jnp.zeros((WS, W, HIDDEN), jnp.float32)
    for b in range(3):
        sb = t[:, b:b + W, :]
        for a in range(3):
            acc = acc + sb[a:a + WS] * wdw[a * 3 + b]
    t1 = acc[..., :DIM]
    t2 = acc[..., DIM:]
    g = 0.5 * t1 * (1.0 + jax.lax.erf(t1 * (2.0 ** -0.5))) * t2
    mlp = jnp.dot(g.reshape(ROWS_PER_STEP, DIM), woutt_ref[...],
                  preferred_element_type=jnp.float32)
    y_ref[...] = (xc.reshape(ROWS_PER_STEP, DIM) + mlp).T


def _mlp(x1_sp, g2, b2, w_int, w_dw9, w_outt):
    clamp = lambda i: jnp.clip(i, 0, NH - 1)
    return pl.pallas_call(
        _mlp_body,
        grid=(NH,),
        in_specs=[
            pl.BlockSpec((WS, W, DIM), lambda i: (clamp(i - 1), 0, 0)),
            pl.BlockSpec((WS, W, DIM), lambda i: (i, 0, 0)),
            pl.BlockSpec((WS, W, DIM), lambda i: (clamp(i + 1), 0, 0)),
            pl.BlockSpec((DIM,), lambda i: (0,)),
            pl.BlockSpec((DIM,), lambda i: (0,)),
            pl.BlockSpec((DIM, HIDDEN), lambda i: (0, 0)),
            pl.BlockSpec((9, HIDDEN), lambda i: (0, 0)),
            pl.BlockSpec((DIM, DIM), lambda i: (0, 0)),
        ],
        out_specs=pl.BlockSpec((DIM, ROWS_PER_STEP), lambda i: (0, i)),
        out_shape=jax.ShapeDtypeStruct((DIM, L), jnp.float32),
    )(x1_sp, x1_sp, x1_sp, g2, b2, w_int, w_dw9, w_outt)


@jax.jit
def kernel(x, g1, b1, wqkv, bqkv, lepe_w, wo, bo, g2, b2, w_in, w_dw, w_out):
    zpad = jnp.zeros((DIM, 32), jnp.float32)
    wqkv_pad = jnp.concatenate(
        [wqkv[:, :DIM], zpad, wqkv[:, DIM:2 * DIM], zpad,
         wqkv[:, 2 * DIM:], zpad], axis=1)
    bqkv_pad = jnp.concatenate(
        [bqkv[:DIM], jnp.zeros((32,), jnp.float32),
         bqkv[DIM:2 * DIM], jnp.zeros((32,), jnp.float32),
         bqkv[2 * DIM:], jnp.zeros((32,), jnp.float32)])

    q_sp, kw, vw, qr, kr = _ln_qkv(x, g1, b1, wqkv_pad, bqkv_pad)

    idx8 = _route(qr, kr)

    lepe_wt = lepe_w.reshape(DIM, 25).T
    x1 = _attn_lepe(idx8, q_sp, kw, vw, x, lepe_wt, wo, bo)

    x1_sp = x1.reshape(H, W, DIM)
    w_int = w_in.T
    w_dw9 = w_dw.reshape(HIDDEN, 9).T
    w_outt = w_out.T
    y = _mlp(x1_sp, g2, b2, w_int, w_dw9, w_outt)

    return y.reshape(1, DIM, H, W)


# speedup vs baseline: 1.1007x; 1.1007x over previous
import functools
import jax
import jax.numpy as jnp
from jax.experimental import pallas as pl
from jax.experimental.pallas import tpu as pltpu

DIM = 96
H = 224
W = 224
WS = 8
HEADS = 8
TOPK = 4
HIDDEN = 192
NH = H // WS
NWIN = NH * NH
WS2 = WS * WS
L = H * W
DH = DIM // HEADS
ROWS_PER_STEP = WS * W


def _ln_qkv_body(x_ref, g1_ref, b1_ref, wqkv_ref, bqkv_ref,
                 qsp_ref, kw_ref, vw_ref, qr_ref, kr_ref):
    x = x_ref[...].reshape(DIM, ROWS_PER_STEP).T
    mu = jnp.mean(x, axis=-1, keepdims=True)
    var = jnp.mean((x - mu) ** 2, axis=-1, keepdims=True)
    y = (x - mu) * jax.lax.rsqrt(var + 1e-5) * g1_ref[...] + b1_ref[...]
    qkv = jnp.dot(y, wqkv_ref[...], preferred_element_type=jnp.float32)
    qkv = qkv + bqkv_ref[...]
    q = qkv[:, :DIM]
    k = qkv[:, 128:128 + DIM]
    v = qkv[:, 256:256 + DIM]
    qsp_ref[...] = q.reshape(WS, W, DIM)
    k4 = k.reshape(WS, NH, WS, DIM)
    v4 = v.reshape(WS, NH, WS, DIM)
    kw_ref[...] = k4.transpose(1, 0, 2, 3).reshape(NH, WS2, DIM)
    vw_ref[...] = v4.transpose(1, 0, 2, 3).reshape(NH, WS2, DIM)
    qr_ref[0] = jnp.mean(q.reshape(WS, NH, WS, DIM), axis=(0, 2))
    kr_ref[0] = jnp.mean(k4, axis=(0, 2))


def _ln_qkv(x, g1, b1, wqkv_pad, bqkv_pad):
    return pl.pallas_call(
        _ln_qkv_body,
        grid=(NH,),
        in_specs=[
            pl.BlockSpec((1, DIM, WS, W), lambda i: (0, 0, i, 0)),
            pl.BlockSpec((DIM,), lambda i: (0,)),
            pl.BlockSpec((DIM,), lambda i: (0,)),
            pl.BlockSpec((DIM, 384), lambda i: (0, 0)),
            pl.BlockSpec((384,), lambda i: (0,)),
        ],
        out_specs=[
            pl.BlockSpec((WS, W, DIM), lambda i: (i, 0, 0)),
            pl.BlockSpec((NH, WS2, DIM), lambda i: (i, 0, 0)),
            pl.BlockSpec((NH, WS2, DIM), lambda i: (i, 0, 0)),
            pl.BlockSpec((1, NH, DIM), lambda i: (i, 0, 0)),
            pl.BlockSpec((1, NH, DIM), lambda i: (i, 0, 0)),
        ],
        out_shape=[
            jax.ShapeDtypeStruct((H, W, DIM), jnp.float32),
            jax.ShapeDtypeStruct((NWIN, WS2, DIM), jnp.float32),
            jax.ShapeDtypeStruct((NWIN, WS2, DIM), jnp.float32),
            jax.ShapeDtypeStruct((NH, NH, DIM), jnp.float32),
            jax.ShapeDtypeStruct((NH, NH, DIM), jnp.float32),
        ],
    )(x, g1, b1, wqkv_pad, bqkv_pad)


def _route_body(qr_ref, kr_ref, idx_ref):
    qr = qr_ref[...].reshape(NWIN, DIM)
    kr = kr_ref[...].reshape(NWIN, DIM)
    a = jnp.dot(qr, kr.T, preferred_element_type=jnp.float32)
    col = jax.lax.broadcasted_iota(jnp.int32, (NWIN, NWIN), 1)
    for t in range(TOPK):
        m = jnp.max(a, axis=-1, keepdims=True)
        amax = jnp.min(jnp.where(a >= m, col, jnp.int32(2 ** 30)),
                       axis=-1, keepdims=True)
        idx_ref[t, :] = amax[:, 0]
        a = jnp.where(col == amax, -jnp.inf, a)


def _route(qr, kr):
    return pl.pallas_call(
        _route_body,
        grid=(1,),
        in_specs=[
            pl.BlockSpec((NH, NH, DIM), lambda i: (0, 0, 0)),
            pl.BlockSpec((NH, NH, DIM), lambda i: (0, 0, 0)),
        ],
        out_specs=pl.BlockSpec((8, NWIN), lambda i: (0, 0)),
        out_shape=jax.ShapeDtypeStruct((8, NWIN), jnp.int32),
    )(qr, kr)


WIN_PER_STEP = 28
SCALE = DH ** -0.5


def _win_rows_to_spatial(t, nrows):
    return (t.reshape(NH, nrows, WS, DIM)
             .transpose(1, 0, 2, 3)
             .reshape(nrows, W, DIM))


def _attn_body(idx_ref, qw_ref, kw_ref, vw_ref, ow_ref):
    step = pl.program_id(0)
    r = jax.lax.broadcasted_iota(jnp.int32, (HEADS * WS2, DIM), 0)
    c = jax.lax.broadcasted_iota(jnp.int32, (HEADS * WS2, DIM), 1)
    head_mask = (r // WS2) == (c // DH)
    for wloc in range(WIN_PER_STEP):
        p = step * NH + wloc
        kparts = []
        vparts = []
        for t in range(TOPK):
            wi = idx_ref[t, p]
            kparts.append(kw_ref[wi])
            vparts.append(vw_ref[wi])
        kg = jnp.concatenate(kparts, axis=0)
        vg = jnp.concatenate(vparts, axis=0)
        q = qw_ref[:, wloc * WS:(wloc + 1) * WS, :].reshape(WS2, DIM)
        q8 = jnp.concatenate([q] * HEADS, axis=0)
        qbd = jnp.where(head_mask, q8, 0.0) * SCALE
        s = jax.lax.dot_general(
            qbd, kg, (((1,), (1,)), ((), ())),
            preferred_element_type=jnp.float32)
        e = jnp.exp(s)
        rden = 1.0 / jnp.sum(e, axis=-1, keepdims=True)
        o_all = jax.lax.dot_general(
            e, vg, (((1,), (0,)), ((), ())),
            preferred_element_type=jnp.float32) * rden
        o = jnp.concatenate(
            [o_all[h * WS2:(h + 1) * WS2, h * DH:(h + 1) * DH]
             for h in range(HEADS)], axis=1)
        ow_ref[:, wloc * WS:(wloc + 1) * WS, :] = o.reshape(WS, WS, DIM)


def _attn(idx8, q_sp, kw, vw):
    return pl.pallas_call(
        _attn_body,
        grid_spec=pltpu.PrefetchScalarGridSpec(
            num_scalar_prefetch=1,
            grid=(NWIN // WIN_PER_STEP,),
            in_specs=[
                pl.BlockSpec((WS, W, DIM), lambda i, idx: (i, 0, 0)),
                pl.BlockSpec((NWIN, WS2, DIM), lambda i, idx: (0, 0, 0)),
                pl.BlockSpec((NWIN, WS2, DIM), lambda i, idx: (0, 0, 0)),
            ],
            out_specs=pl.BlockSpec((WS, W, DIM), lambda i, idx: (i, 0, 0)),
        ),
        out_shape=jax.ShapeDtypeStruct((H, W, DIM), jnp.float32),
        compiler_params=pltpu.CompilerParams(
            dimension_semantics=("arbitrary",)),
    )(idx8, q_sp, kw, vw)


def _lepe_proj_body(o_ref, vw_ref, x_ref, lw_ref, wo_ref, bo_ref, x1_ref):
    i = pl.program_id(0)
    o_sp = o_ref[...]
    vc = _win_rows_to_spatial(
        vw_ref[pl.ds(i * NH, NH)], WS)
    vp = _win_rows_to_spatial(
        vw_ref[pl.ds(jnp.maximum(i - 1, 0) * NH, NH)][:, WS2 - 2 * WS:, :], 2)
    vn = _win_rows_to_spatial(
        vw_ref[pl.ds(jnp.minimum(i + 1, NH - 1) * NH, NH)][:, :2 * WS, :], 2)
    top = jnp.where(i > 0, vp, 0.0)
    bot = jnp.where(i < NH - 1, vn, 0.0)
    vpad = jnp.concatenate([top, vc, bot], axis=0)
    zc = jnp.zeros((WS + 4, 2, DIM), jnp.float32)
    vpad = jnp.concatenate([zc, vpad, zc], axis=1)
    lw = lw_ref[...]
    lepe = jnp.zeros((WS, W, DIM), jnp.float32)
    for b in range(5):
        sb = vpad[:, b:b + W, :]
        for a in range(5):
            lepe = lepe + sb[a:a + WS] * lw[a * 5 + b]
    o = o_sp + lepe
    y = jnp.dot(o.reshape(ROWS_PER_STEP, DIM), wo_ref[...],
                preferred_element_type=jnp.float32) + bo_ref[...]
    xl = x_ref[...].reshape(DIM, ROWS_PER_STEP).T
    x1_ref[...] = xl + y


def _lepe_proj(o_sp, vw, x, lepe_wt, wo, bo):
    return pl.pallas_call(
        _lepe_proj_body,
        grid=(NH,),
        in_specs=[
            pl.BlockSpec((WS, W, DIM), lambda i: (i, 0, 0)),
            pl.BlockSpec((NWIN, WS2, DIM), lambda i: (0, 0, 0)),
            pl.BlockSpec((1, DIM, WS, W), lambda i: (0, 0, i, 0)),
            pl.BlockSpec((25, DIM), lambda i: (0, 0)),
            pl.BlockSpec((DIM, DIM), lambda i: (0, 0)),
            pl.BlockSpec((DIM,), lambda i: (0,)),
        ],
        out_specs=pl.BlockSpec((ROWS_PER_STEP, DIM), lambda i: (i, 0)),
        out_shape=jax.ShapeDtypeStruct((L, DIM), jnp.float32),
    )(o_sp, vw, x, lepe_wt, wo, bo)


def _mlp_body(xp_ref, xc_ref, xn_ref, g2_ref, b2_ref,
              wint_ref, wdw_ref, woutt_ref, y_ref):
    i = pl.program_id(0)
    xc = xc_ref[...]
    xh = jnp.concatenate(
        [xp_ref[WS - 1:WS], xc, xn_ref[0:1]], axis=0)
    mu = jnp.mean(xh, axis=-1, keepdims=True)
    var = jnp.mean((xh - mu) ** 2, axis=-1, keepdims=True)
    y2 = (xh - mu) * jax.lax.rsqrt(var + 1e-5) * g2_ref[...] + b2_ref[...]
    t = jnp.dot(y2.reshape((WS + 2) * W, DIM), wint_ref[...],
                preferred_element_type=jnp.float32)
    t = t.reshape(WS + 2, W, HIDDEN)
    row = jax.lax.broadcasted_iota(jnp.int32, (WS + 2, 1, 1), 0)
    valid = jnp.logical_and(
        jnp.logical_or(i > 0, row > 0),
        jnp.logical_or(i < NH - 1, row < WS + 1))
    t = jnp.where(valid, t, 0.0)
    zc = jnp.zeros((WS + 2, 1, HIDDEN), jnp.float32)
    t = jnp.concatenate([zc, t, zc], axis=1)
    wdw = wdw_ref[...]
    acc = jnp.zeros((WS, W, HIDDEN), jnp.float32)
    for b in range(3):
        sb = t[:, b:b + W, :]
        for a in range(3):
            acc = acc + sb[a:a + WS] * wdw[a * 3 + b]
    t1 = acc[..., :DIM]
    t2 = acc[..., DIM:]
    g = 0.5 * t1 * (1.0 + jax.lax.erf(t1 * (2.0 ** -0.5))) * t2
    mlp = jnp.dot(g.reshape(ROWS_PER_STEP, DIM), woutt_ref[...],
                  preferred_element_type=jnp.float32)
    y_ref[...] = (xc.reshape(ROWS_PER_STEP, DIM) + mlp).T


def _mlp(x1_sp, g2, b2, w_int, w_dw9, w_outt):
    clamp = lambda i: jnp.clip(i, 0, NH - 1)
    return pl.pallas_call(
        _mlp_body,
        grid=(NH,),
        in_specs=[
            pl.BlockSpec((WS, W, DIM), lambda i: (clamp(i - 1), 0, 0)),
            pl.BlockSpec((WS, W, DIM), lambda i: (i, 0, 0)),
            pl.BlockSpec((WS, W, DIM), lambda i: (clamp(i + 1), 0, 0)),
            pl.BlockSpec((DIM,), lambda i: (0,)),
            pl.BlockSpec((DIM,), lambda i: (0,)),
            pl.BlockSpec((DIM, HIDDEN), lambda i: (0, 0)),
            pl.BlockSpec((9, HIDDEN), lambda i: (0, 0)),
            pl.BlockSpec((DIM, DIM), lambda i: (0, 0)),
        ],
        out_specs=pl.BlockSpec((DIM, ROWS_PER_STEP), lambda i: (0, i)),
        out_shape=jax.ShapeDtypeStruct((DIM, L), jnp.float32),
    )(x1_sp, x1_sp, x1_sp, g2, b2, w_int, w_dw9, w_outt)


@jax.jit
def kernel(x, g1, b1, wqkv, bqkv, lepe_w, wo, bo, g2, b2, w_in, w_dw, w_out):
    zpad = jnp.zeros((DIM, 32), jnp.float32)
    wqkv_pad = jnp.concatenate(
        [wqkv[:, :DIM], zpad, wqkv[:, DIM:2 * DIM], zpad,
         wqkv[:, 2 * DIM:], zpad], axis=1)
    bqkv_pad = jnp.concatenate(
        [bqkv[:DIM], jnp.zeros((32,), jnp.float32),
         bqkv[DIM:2 * DIM], jnp.zeros((32,), jnp.float32),
         bqkv[2 * DIM:], jnp.zeros((32,), jnp.float32)])

    q_sp, kw, vw, qr, kr = _ln_qkv(x, g1, b1, wqkv_pad, bqkv_pad)

    idx8 = _route(qr, kr)

    o_sp = _attn(idx8, q_sp, kw, vw)

    lepe_wt = lepe_w.reshape(DIM, 25).T
    x1 = _lepe_proj(o_sp, vw, x, lepe_wt, wo, bo)

    x1_sp = x1.reshape(H, W, DIM)
    w_int = w_in.T
    w_dw9 = w_dw.reshape(HIDDEN, 9).T
    w_outt = w_out.T
    y = _mlp(x1_sp, g2, b2, w_int, w_dw9, w_outt)

    return y.reshape(1, DIM, H, W)
